# baseline (speedup 1.0000x reference)
import functools

import jax
import jax.numpy as jnp
from jax import lax
from jax.experimental import pallas as pl
from jax.experimental.pallas import tpu as pltpu
from jax.experimental.pallas import tpu_sc as plsc

_GATHER_WINDOW = 128
_TB = 1024
_PACK_CHUNK = 4000
_PREC = lax.Precision.DEFAULT


def _linearize_tables_tc(tablesT):
    f, d, v = tablesT.shape
    rows_per_blk = v * d // 128
    chunk = _PACK_CHUNK
    bounds = list(range(0, v, chunk))

    quarter = chunk // 4

    def body(in_ref, out_ref):
        for c0 in bounds:
            slab = jnp.concatenate(
                [in_ref[0, :, c0 + k * quarter:c0 + (k + 1) * quarter]
                 for k in range(4)], axis=0)
            out_ref[c0 // 4:(c0 + chunk) // 4, :] = slab.T

    return pl.pallas_call(
        body,
        grid=(f,),
        in_specs=[pl.BlockSpec((1, d, v), lambda i: (i, 0, 0))],
        out_specs=pl.BlockSpec((rows_per_blk, 128), lambda i: (i, 0)),
        out_shape=jax.ShapeDtypeStruct((f * v * d // 128, 128), jnp.float32),
        compiler_params=pltpu.CompilerParams(
            dimension_semantics=("parallel",),
            vmem_limit_bytes=130 * 1024 * 1024),
    )(tablesT)


def _gather_sc(tables_flat, idx):
    num_idx = idx.shape[0]
    d = tables_flat.shape[1]
    idx2 = idx.reshape(1, num_idx)
    mesh = plsc.VectorSubcoreMesh(core_axis_name="c", subcore_axis_name="s")

    @functools.partial(
        pl.kernel,
        out_type=jax.ShapeDtypeStruct((num_idx, d), tables_flat.dtype),
        mesh=mesh,
        compiler_params=pltpu.CompilerParams(use_tc_tiling_on_sc=False),
    )
    def gather_kernel(x_hbm, i_hbm, o_hbm):
        def body(i_vmem, o_vmem):
            pltpu.sync_copy(x_hbm.at[i_vmem.at[0]], o_vmem)

        pltpu.emit_pipeline(
            body,
            grid=(num_idx // _GATHER_WINDOW,),
            in_specs=[pl.BlockSpec((1, _GATHER_WINDOW), lambda i: (0, i))],
            out_specs=[pl.BlockSpec((_GATHER_WINDOW, d), lambda i: (i, 0))],
            core_axis_name=("c", "s"),
            dimension_semantics=(pltpu.PARALLEL,),
        )(i_hbm, o_hbm)

    return gather_kernel(tables_flat, idx2)


def _ln_gelu(h, g, bt):
    mu = jnp.mean(h, axis=-1, keepdims=True)
    var = jnp.mean((h - mu) ** 2, axis=-1, keepdims=True)
    h = (h - mu) / jnp.sqrt(var + 1e-5) * g + bt
    return h * 0.5 * (1.0 + lax.erf(h * (2.0 ** -0.5)))


def _mlp_body(emb_ref, xnum_ref, w1a_ref, w1b_ref, b1_ref, g1_ref, bt1_ref,
              w2_ref, b2_ref, g2_ref, bt2_ref, w3_ref, b3_ref, g3_ref, bt3_ref,
              w4_ref, b4_ref, out_ref):
    dot = functools.partial(jnp.dot, preferred_element_type=jnp.float32,
                            precision=_PREC)
    e3 = emb_ref[...].reshape(_TB, 7, 128)
    h = dot(xnum_ref[...], w1b_ref[...])
    for r in range(7):
        h = h + dot(e3[:, r, :], w1a_ref[128 * r:128 * (r + 1), :])
    h = _ln_gelu(h + b1_ref[...], g1_ref[...], bt1_ref[...])
    h = _ln_gelu(dot(h, w2_ref[...]) + b2_ref[...], g2_ref[...], bt2_ref[...])
    h = _ln_gelu(dot(h, w3_ref[...]) + b3_ref[...], g3_ref[...], bt3_ref[...])
    z = dot(h, w4_ref[...]) + b4_ref[...]
    out_ref[...] = jax.nn.sigmoid(z)


def _mlp_tc(embv, xnum, w1a, w1b, b1, g1, bt1, w2, b2, g2, bt2,
            w3, b3, g3, bt3, w4, b4):
    bsz = xnum.shape[0]
    grid = (bsz // _TB,)

    def tile(r):
        return pl.BlockSpec((_TB, r.shape[1]), lambda i: (i, 0))

    def full(r):
        return pl.BlockSpec(r.shape, lambda i: (0, 0))

    emb_spec = pl.BlockSpec((_TB * 7, 128), lambda i: (i, 0))
    return pl.pallas_call(
        _mlp_body,
        grid=grid,
        in_specs=[emb_spec, tile(xnum)] + [full(r) for r in (
            w1a, w1b, b1, g1, bt1, w2, b2, g2, bt2, w3, b3, g3, bt3, w4, b4)],
        out_specs=pl.BlockSpec((_TB, 1), lambda i: (i, 0)),
        out_shape=jax.ShapeDtypeStruct((bsz, 1), jnp.float32),
        compiler_params=pltpu.CompilerParams(
            dimension_semantics=("parallel",)),
    )(embv, xnum, w1a, w1b, b1, g1, bt1, w2, b2, g2, bt2,
      w3, b3, g3, bt3, w4, b4)


def kernel(x_cat, x_num, tables, W1, b1, g1, bt1, W2, b2, g2, bt2,
           W3, b3, g3, bt3, W4, b4):
    f, v, d = tables.shape
    b = x_cat.shape[0]
    tables_flat = _linearize_tables_tc(
        jnp.swapaxes(tables, 1, 2)).reshape(f * v, d)
    offsets = (jnp.arange(f, dtype=jnp.int32) * v)[None, :]
    q = _PACK_CHUNK // 4
    pos = x_cat % _PACK_CHUNK
    perm = (x_cat - pos) + 4 * (pos % q) + pos // q
    idxp = jnp.concatenate(
        [perm + offsets, jnp.zeros((b, 2), jnp.int32)], axis=1).reshape(-1)
    embv = _gather_sc(tables_flat, idxp).reshape(b * 7, 128)
    w1a = jnp.concatenate(
        [W1[: f * d], jnp.zeros((896 - f * d, W1.shape[1]), W1.dtype)])
    w1b = W1[f * d:]
    row = lambda x: x.reshape(1, -1)
    return _mlp_tc(embv, x_num, w1a, w1b, row(b1), row(g1), row(bt1),
                   W2, row(b2), row(g2), row(bt2),
                   W3, row(b3), row(g3), row(bt3), W4, row(b4))


# speedup vs baseline: 1.7080x; 1.7080x over previous
import functools

import jax
import jax.numpy as jnp
from jax import lax
from jax.experimental import pallas as pl
from jax.experimental.pallas import tpu as pltpu
from jax.experimental.pallas import tpu_sc as plsc

_GATHER_WINDOW = 128
_TB = 1024
_PACK_CHUNK = 4000
_PREC = lax.Precision.DEFAULT


def _linearize_tables_tc(tablesT):
    f, d, v = tablesT.shape
    rows_per_blk = v * d // 128
    chunk = _PACK_CHUNK
    bounds = list(range(0, v, chunk))

    quarter = chunk // 4

    def body(in_ref, out_ref):
        for c0 in bounds:
            slab = jnp.concatenate(
                [in_ref[0, :, c0 + k * quarter:c0 + (k + 1) * quarter]
                 for k in range(4)], axis=0)
            out_ref[c0 // 4:(c0 + chunk) // 4, :] = slab.T

    return pl.pallas_call(
        body,
        grid=(f,),
        in_specs=[pl.BlockSpec((1, d, v), lambda i: (i, 0, 0))],
        out_specs=pl.BlockSpec((rows_per_blk, 128), lambda i: (i, 0)),
        out_shape=jax.ShapeDtypeStruct((f * v * d // 128, 128), jnp.float32),
        compiler_params=pltpu.CompilerParams(
            dimension_semantics=("parallel",),
            vmem_limit_bytes=130 * 1024 * 1024),
    )(tablesT)


def _gather_sc(tables_flat, idx):
    num_idx = idx.shape[0]
    d = tables_flat.shape[1]
    idx2 = idx.reshape(1, num_idx)
    mesh = plsc.VectorSubcoreMesh(core_axis_name="c", subcore_axis_name="s")

    @functools.partial(
        pl.kernel,
        out_type=jax.ShapeDtypeStruct((num_idx, d), tables_flat.dtype),
        mesh=mesh,
        compiler_params=pltpu.CompilerParams(use_tc_tiling_on_sc=False),
    )
    def gather_kernel(x_hbm, i_hbm, o_hbm):
        def body(i_vmem, o_vmem):
            pltpu.sync_copy(x_hbm.at[i_vmem.at[0]], o_vmem)

        pltpu.emit_pipeline(
            body,
            grid=(num_idx // _GATHER_WINDOW,),
            in_specs=[pl.BlockSpec((1, _GATHER_WINDOW), lambda i: (0, i))],
            out_specs=[pl.BlockSpec((_GATHER_WINDOW, d), lambda i: (i, 0))],
            core_axis_name=("c", "s"),
            dimension_semantics=(pltpu.PARALLEL,),
        )(i_hbm, o_hbm)

    return gather_kernel(tables_flat, idx2)


def _ln_gelu(h, g, bt):
    mu = jnp.mean(h, axis=-1, keepdims=True)
    var = jnp.mean((h - mu) ** 2, axis=-1, keepdims=True)
    h = (h - mu) / jnp.sqrt(var + 1e-5) * g + bt
    return h * 0.5 * (1.0 + lax.erf(h * (2.0 ** -0.5)))


def _mlp_body(emb_ref, xnum_ref, w1a_ref, w1b_ref, b1_ref, g1_ref, bt1_ref,
              w2_ref, b2_ref, g2_ref, bt2_ref, w3_ref, b3_ref, g3_ref, bt3_ref,
              w4_ref, b4_ref, out_ref):
    dot = functools.partial(jnp.dot, preferred_element_type=jnp.float32,
                            precision=_PREC)
    e3 = emb_ref[...].reshape(_TB, 7, 128)
    h = dot(xnum_ref[...], w1b_ref[...])
    for r in range(7):
        h = h + dot(e3[:, r, :], w1a_ref[128 * r:128 * (r + 1), :])
    h = _ln_gelu(h + b1_ref[...], g1_ref[...], bt1_ref[...])
    h = _ln_gelu(dot(h, w2_ref[...]) + b2_ref[...], g2_ref[...], bt2_ref[...])
    h = _ln_gelu(dot(h, w3_ref[...]) + b3_ref[...], g3_ref[...], bt3_ref[...])
    z = dot(h, w4_ref[...]) + b4_ref[...]
    out_ref[...] = jax.nn.sigmoid(z)


def _mlp_tc(embv, xnum, w1a, w1b, b1, g1, bt1, w2, b2, g2, bt2,
            w3, b3, g3, bt3, w4, b4):
    bsz = xnum.shape[0]
    grid = (bsz // _TB,)

    def tile(r):
        return pl.BlockSpec((_TB, r.shape[1]), lambda i: (i, 0))

    def full(r):
        return pl.BlockSpec(r.shape, lambda i: (0, 0))

    emb_spec = pl.BlockSpec((_TB * 7, 128), lambda i: (i, 0))
    return pl.pallas_call(
        _mlp_body,
        grid=grid,
        in_specs=[emb_spec, tile(xnum)] + [full(r) for r in (
            w1a, w1b, b1, g1, bt1, w2, b2, g2, bt2, w3, b3, g3, bt3, w4, b4)],
        out_specs=pl.BlockSpec((_TB, 1), lambda i: (i, 0)),
        out_shape=jax.ShapeDtypeStruct((bsz, 1), jnp.float32),
        compiler_params=pltpu.CompilerParams(
            dimension_semantics=("parallel",)),
    )(embv, xnum, w1a, w1b, b1, g1, bt1, w2, b2, g2, bt2,
      w3, b3, g3, bt3, w4, b4)


def kernel(x_cat, x_num, tables, W1, b1, g1, bt1, W2, b2, g2, bt2,
           W3, b3, g3, bt3, W4, b4):
    f, v, d = tables.shape
    b = x_cat.shape[0]
    tables_flat = _linearize_tables_tc(
        jnp.swapaxes(tables, 1, 2)).reshape(f * v, d)
    offsets = (jnp.arange(f, dtype=jnp.int32) * v)[None, :]
    q = _PACK_CHUNK // 4
    pos = x_cat % _PACK_CHUNK
    perm = (x_cat - pos) + 4 * (pos % q) + pos // q
    gidx = perm + offsets
    idxp = jnp.concatenate([gidx, gidx[:, :2]], axis=1).reshape(-1)
    embv = _gather_sc(tables_flat, idxp).reshape(b * 7, 128)
    w1a = jnp.concatenate(
        [W1[: f * d], jnp.zeros((896 - f * d, W1.shape[1]), W1.dtype)])
    w1b = W1[f * d:]
    row = lambda x: x.reshape(1, -1)
    return _mlp_tc(embv, x_num, w1a, w1b, row(b1), row(g1), row(bt1),
                   W2, row(b2), row(g2), row(bt2),
                   W3, row(b3), row(g3), row(bt3), W4, row(b4))


# speedup vs baseline: 1.7553x; 1.0277x over previous
import functools

import jax
import jax.numpy as jnp
from jax import lax
from jax.experimental import pallas as pl
from jax.experimental.pallas import tpu as pltpu
from jax.experimental.pallas import tpu_sc as plsc

_GATHER_WINDOW = 128
_TB = 1024
_PACK_CHUNK = 20000
_PREC = lax.Precision.DEFAULT


def _linearize_tables_tc(tablesT):
    f, d, v = tablesT.shape
    rows_per_blk = v * d // 128
    chunk = _PACK_CHUNK
    bounds = list(range(0, v, chunk))

    quarter = chunk // 4

    def body(in_ref, out_ref):
        for c0 in bounds:
            slab = jnp.concatenate(
                [in_ref[0, :, c0 + k * quarter:c0 + (k + 1) * quarter]
                 for k in range(4)], axis=0)
            out_ref[c0 // 4:(c0 + chunk) // 4, :] = slab.T

    return pl.pallas_call(
        body,
        grid=(f,),
        in_specs=[pl.BlockSpec((1, d, v), lambda i: (i, 0, 0))],
        out_specs=pl.BlockSpec((rows_per_blk, 128), lambda i: (i, 0)),
        out_shape=jax.ShapeDtypeStruct((f * v * d // 128, 128), jnp.float32),
        compiler_params=pltpu.CompilerParams(
            dimension_semantics=("parallel",),
            vmem_limit_bytes=130 * 1024 * 1024),
    )(tablesT)


def _gather_sc(tables_flat, idx):
    num_idx = idx.shape[0]
    d = tables_flat.shape[1]
    idx2 = idx.reshape(1, num_idx)
    mesh = plsc.VectorSubcoreMesh(core_axis_name="c", subcore_axis_name="s")

    @functools.partial(
        pl.kernel,
        out_type=jax.ShapeDtypeStruct((num_idx, d), tables_flat.dtype),
        mesh=mesh,
        compiler_params=pltpu.CompilerParams(use_tc_tiling_on_sc=False),
    )
    def gather_kernel(x_hbm, i_hbm, o_hbm):
        def body(i_vmem, o_vmem):
            pltpu.sync_copy(x_hbm.at[i_vmem.at[0]], o_vmem)

        pltpu.emit_pipeline(
            body,
            grid=(num_idx // _GATHER_WINDOW,),
            in_specs=[pl.BlockSpec((1, _GATHER_WINDOW), lambda i: (0, i))],
            out_specs=[pl.BlockSpec((_GATHER_WINDOW, d), lambda i: (i, 0))],
            core_axis_name=("c", "s"),
            dimension_semantics=(pltpu.PARALLEL,),
        )(i_hbm, o_hbm)

    return gather_kernel(tables_flat, idx2)


def _ln_gelu(h, g, bt):
    mu = jnp.mean(h, axis=-1, keepdims=True)
    var = jnp.mean((h - mu) ** 2, axis=-1, keepdims=True)
    h = (h - mu) / jnp.sqrt(var + 1e-5) * g + bt
    return h * 0.5 * (1.0 + lax.erf(h * (2.0 ** -0.5)))


def _mlp_body(emb_ref, xnum_ref, w1a_ref, w1b_ref, b1_ref, g1_ref, bt1_ref,
              w2_ref, b2_ref, g2_ref, bt2_ref, w3_ref, b3_ref, g3_ref, bt3_ref,
              w4_ref, b4_ref, out_ref):
    dot = functools.partial(jnp.dot, preferred_element_type=jnp.float32,
                            precision=_PREC)
    e3 = emb_ref[...].reshape(_TB, 7, 128)
    h = dot(xnum_ref[...], w1b_ref[...])
    for r in range(7):
        h = h + dot(e3[:, r, :], w1a_ref[128 * r:128 * (r + 1), :])
    h = _ln_gelu(h + b1_ref[...], g1_ref[...], bt1_ref[...])
    h = _ln_gelu(dot(h, w2_ref[...]) + b2_ref[...], g2_ref[...], bt2_ref[...])
    h = _ln_gelu(dot(h, w3_ref[...]) + b3_ref[...], g3_ref[...], bt3_ref[...])
    z = dot(h, w4_ref[...]) + b4_ref[...]
    out_ref[...] = jax.nn.sigmoid(z)


def _mlp_tc(embv, xnum, w1a, w1b, b1, g1, bt1, w2, b2, g2, bt2,
            w3, b3, g3, bt3, w4, b4):
    bsz = xnum.shape[0]
    grid = (bsz // _TB,)

    def tile(r):
        return pl.BlockSpec((_TB, r.shape[1]), lambda i: (i, 0))

    def full(r):
        return pl.BlockSpec(r.shape, lambda i: (0, 0))

    emb_spec = pl.BlockSpec((_TB * 7, 128), lambda i: (i, 0))
    return pl.pallas_call(
        _mlp_body,
        grid=grid,
        in_specs=[emb_spec, tile(xnum)] + [full(r) for r in (
            w1a, w1b, b1, g1, bt1, w2, b2, g2, bt2, w3, b3, g3, bt3, w4, b4)],
        out_specs=pl.BlockSpec((_TB, 1), lambda i: (i, 0)),
        out_shape=jax.ShapeDtypeStruct((bsz, 1), jnp.float32),
        compiler_params=pltpu.CompilerParams(
            dimension_semantics=("parallel",)),
    )(embv, xnum, w1a, w1b, b1, g1, bt1, w2, b2, g2, bt2,
      w3, b3, g3, bt3, w4, b4)


def kernel(x_cat, x_num, tables, W1, b1, g1, bt1, W2, b2, g2, bt2,
           W3, b3, g3, bt3, W4, b4):
    f, v, d = tables.shape
    b = x_cat.shape[0]
    tables_flat = _linearize_tables_tc(
        jnp.swapaxes(tables, 1, 2)).reshape(f * v, d)
    offsets = (jnp.arange(f, dtype=jnp.int32) * v)[None, :]
    q = _PACK_CHUNK // 4
    pos = x_cat % _PACK_CHUNK
    perm = (x_cat - pos) + 4 * (pos % q) + pos // q
    gidx = perm + offsets
    idxp = jnp.concatenate([gidx, gidx[:, :2]], axis=1).reshape(-1)
    embv = _gather_sc(tables_flat, idxp).reshape(b * 7, 128)
    w1a = jnp.concatenate(
        [W1[: f * d], jnp.zeros((896 - f * d, W1.shape[1]), W1.dtype)])
    w1b = W1[f * d:]
    row = lambda x: x.reshape(1, -1)
    return _mlp_tc(embv, x_num, w1a, w1b, row(b1), row(g1), row(bt1),
                   W2, row(b2), row(g2), row(bt2),
                   W3, row(b3), row(g3), row(bt3), W4, row(b4))
